# baseline (speedup 1.0000x reference)
import functools

import jax
import jax.numpy as jnp
from jax import lax
from jax.experimental import pallas as pl
from jax.experimental.pallas import tpu as pltpu
from jax.experimental.pallas import tpu_sc as plsc

K = 16
TS = 8
ROWW = 16


def _cell(q0_ref, keys_ref,
          dist_ref, idx_ref, outx_ref,
          xcur_ref):
    s = pl.program_id(1)
    nk = keys_ref.shape[2]
    nq = q0_ref.shape[2]

    @pl.when(s == 0)
    def _init_chain():
        xcur_ref[0:3, :] = q0_ref[0]

    keys = keys_ref[0, 0]
    kx = keys[:, 0:1]
    ky = keys[:, 1:2]
    kz = keys[:, 2:3]
    dx = xcur_ref[0:1, :] - kx
    dy = xcur_ref[1:2, :] - ky
    dz = xcur_ref[2:3, :] - kz
    d2 = dx * dx + dy * dy + dz * dz

    iota = jax.lax.broadcasted_iota(jnp.int32, (nk, nq), 0)

    pv = jnp.full((1, nq), -1.0, jnp.float32)
    pi = jnp.full((1, nq), -1, jnp.int32)
    for r in range(K):
        live = jnp.logical_or(d2 > pv, jnp.logical_and(d2 == pv, iota > pi))
        d2m = jnp.where(live, d2, jnp.float32(jnp.inf))
        minv = jnp.min(d2m, axis=0, keepdims=True)
        cand = jnp.where(d2m == minv, iota, jnp.int32(nk))
        mini = jnp.min(cand, axis=0, keepdims=True)
        dist_ref[0, 0, r, :] = minv[0]
        idx_ref[0, 0, r, :] = mini[0]
        if r == 0:
            onehot0 = iota == mini
        pv, pi = minv, mini

    zero = jnp.float32(0.0)
    gx = jnp.sum(jnp.where(onehot0, kx, zero), axis=0, keepdims=True)
    gy = jnp.sum(jnp.where(onehot0, ky, zero), axis=0, keepdims=True)
    gz = jnp.sum(jnp.where(onehot0, kz, zero), axis=0, keepdims=True)
    xcur_ref[0:1, :] = gx
    xcur_ref[1:2, :] = gy
    xcur_ref[2:3, :] = gz
    outx_ref[0, 0, 0:1, :] = gx
    outx_ref[0, 0, 1:2, :] = gy
    outx_ref[0, 0, 2:3, :] = gz


def _sc_gather_call(table, flat_idx):
    total = flat_idx.shape[0]
    info = plsc.get_sparse_core_info()
    nw = info.num_cores * info.num_subcores
    kch = 16
    ch = kch * 128
    per_w = total // nw
    n_iter = per_w // ch
    assert per_w % ch == 0
    idx2d = flat_idx.reshape(total // 128, 128)

    mesh = plsc.VectorSubcoreMesh(core_axis_name="c", subcore_axis_name="s")

    @functools.partial(
        pl.kernel,
        mesh=mesh,
        out_type=jax.ShapeDtypeStruct((total, ROWW), jnp.float32),
        compiler_params=pltpu.CompilerParams(use_tc_tiling_on_sc=False),
        scratch_types=[
            pltpu.VMEM((kch, 128), jnp.int32),
            pltpu.VMEM((ch, ROWW), jnp.float32),
            pltpu.SemaphoreType.DMA,
        ],
    )
    def _k(table_hbm, idx_hbm, out_hbm, idx_v, rows_v, sem):
        wid = lax.axis_index("s") * info.num_cores + lax.axis_index("c")
        base = wid * per_w

        def body(i, carry):
            off = pl.multiple_of(base + i * ch, ch)
            pltpu.sync_copy(
                idx_hbm.at[pl.ds(pl.multiple_of(off // 128, kch), kch), :],
                idx_v)
            copies = []
            for j in range(kch):
                copies.append(pltpu.async_copy(
                    table_hbm.at[idx_v.at[j]],
                    rows_v.at[pl.ds(j * 128, 128), :],
                    sem,
                ))
            for cp in copies:
                cp.wait()
            pltpu.sync_copy(rows_v, out_hbm.at[pl.ds(off, ch), :])
            return carry

        lax.fori_loop(0, n_iter, body, 0)

    return _k(table, idx2d)


def kernel(point_seq):
    b, t, n, d = point_seq.shape
    assert t % TS == 0 and d == 3
    nseg = t // TS
    nq = n // 2
    nprob = 2 * b * nseg

    perm_key = jax.random.key(42)
    ridx = jnp.stack([
        jax.random.permutation(jax.random.fold_in(perm_key, i), n)[:nq]
        for i in range(nseg)
    ])

    b_arr = jnp.tile(jnp.repeat(jnp.arange(b), nseg), 2)
    seg_arr = jnp.tile(jnp.arange(nseg), 2 * b)
    dir_arr = jnp.repeat(jnp.arange(2), b * nseg)
    f0_arr = seg_arr * TS + jnp.where(dir_arr == 0, 0, TS - 1)

    q0 = point_seq[b_arr[:, None], f0_arr[:, None], ridx[seg_arr], :]
    q0 = jnp.transpose(q0, (0, 2, 1))

    bn = b * nseg

    def _key_map(p, s):
        dir_ = p // bn
        rem = p % bn
        b_ = rem // nseg
        seg = rem % nseg
        local = jnp.where(dir_ == 0, s, TS - 1 - s)
        return (b_, seg * TS + local, 0, 0)

    out_shapes = (
        jax.ShapeDtypeStruct((nprob, TS, K, nq), jnp.float32),
        jax.ShapeDtypeStruct((nprob, TS, K, nq), jnp.int32),
        jax.ShapeDtypeStruct((nprob, TS, 3, nq), jnp.float32),
    )

    grid = (nprob, TS)
    dist_all, idx_all, outx_all = pl.pallas_call(
        _cell,
        grid=grid,
        in_specs=[
            pl.BlockSpec((1, 3, nq), lambda p, s: (p, 0, 0)),
            pl.BlockSpec((1, 1, n, 3), _key_map),
        ],
        out_specs=[
            pl.BlockSpec((1, 1, K, nq), lambda p, s: (p, s, 0, 0)),
            pl.BlockSpec((1, 1, K, nq), lambda p, s: (p, s, 0, 0)),
            pl.BlockSpec((1, 1, 3, nq), lambda p, s: (p, s, 0, 0)),
        ],
        out_shape=out_shapes,
        scratch_shapes=[
            pltpu.VMEM((8, nq), jnp.float32),
        ],
        compiler_params=pltpu.CompilerParams(
            dimension_semantics=("parallel", "arbitrary"),
        ),
    )(q0, point_seq)

    def _split(x):
        x = x.reshape(2, b, nseg, *x.shape[1:])
        return x[0], x[1]

    def _asm_kq(x):
        f, bk = _split(x)
        bk = jnp.flip(bk, axis=2)
        y = jnp.concatenate([f, bk], axis=-1)
        return y.transpose(0, 1, 2, 4, 3).reshape(b, t, n, K)

    dist = _asm_kq(dist_all)
    idx = _asm_kq(idx_all)

    t_iota = jnp.arange(t, dtype=jnp.int32).reshape(1, t, 1, 1)
    b_iota = jnp.arange(b, dtype=jnp.int32).reshape(b, 1, 1, 1)
    n_iota = jnp.arange(n, dtype=jnp.int32).reshape(1, 1, n, 1)
    seg_base = (t_iota // TS) * TS
    featframe = jnp.where(
        n_iota < nq,
        jnp.maximum(t_iota - 1, seg_base),
        jnp.minimum(t_iota + 1, seg_base + TS - 1),
    )
    pts_rows = (b_iota * t + t_iota) * n + idx
    fe_rows = (b_iota * t + featframe) * n + idx
    flat_rows = jnp.concatenate(
        [pts_rows.reshape(-1), fe_rows.reshape(-1)])
    table = jnp.pad(point_seq.reshape(b * t * n, 3),
                    ((0, 0), (0, ROWW - 3)))
    gathered = _sc_gather_call(table, flat_rows)
    half = pts_rows.size
    pts = gathered[:half, 0:3].reshape(b, t, n, K, 3)
    pfe = gathered[half:, 0:3].reshape(b, t, n, K, 3)

    f, bk = _split(outx_all)
    outx = jnp.concatenate([f, bk], axis=-1)
    outx = outx.transpose(0, 1, 2, 4, 3).reshape(b, t, n, 3)

    return pts, pfe, dist, idx, idx, outx


# speedup vs baseline: 1.3014x; 1.3014x over previous
import functools

import jax
import jax.numpy as jnp
from jax import lax
from jax.experimental import pallas as pl
from jax.experimental.pallas import tpu as pltpu
from jax.experimental.pallas import tpu_sc as plsc

K = 16
TS = 8
ROWW = 16


def _cell(q0_ref, keys_ref,
          dist_ref, idx_ref, outx_ref,
          d2_ref, xcur_ref):
    s = pl.program_id(1)
    nk = keys_ref.shape[2]
    nq = q0_ref.shape[2]

    @pl.when(s == 0)
    def _init_chain():
        xcur_ref[0:3, :] = q0_ref[0]

    keys = keys_ref[0, 0]
    kx = keys[:, 0:1]
    ky = keys[:, 1:2]
    kz = keys[:, 2:3]
    dx = xcur_ref[0:1, :] - kx
    dy = xcur_ref[1:2, :] - ky
    dz = xcur_ref[2:3, :] - kz
    d2_ref[...] = dx * dx + dy * dy + dz * dz

    iota = jax.lax.broadcasted_iota(jnp.int32, (nk, nq), 0)

    for r in range(K):
        d2 = d2_ref[...]
        minv = jnp.min(d2, axis=0, keepdims=True)
        cand = jnp.where(d2 == minv, iota, jnp.int32(nk))
        mini = jnp.min(cand, axis=0, keepdims=True)
        if r < K - 1:
            d2_ref[...] = jnp.where(iota == mini, jnp.float32(jnp.inf), d2)
        dist_ref[0, 0, r, :] = minv[0]
        idx_ref[0, 0, r, :] = mini[0]
        if r == 0:
            onehot0 = iota == mini

    zero = jnp.float32(0.0)
    gx = jnp.sum(jnp.where(onehot0, kx, zero), axis=0, keepdims=True)
    gy = jnp.sum(jnp.where(onehot0, ky, zero), axis=0, keepdims=True)
    gz = jnp.sum(jnp.where(onehot0, kz, zero), axis=0, keepdims=True)
    xcur_ref[0:1, :] = gx
    xcur_ref[1:2, :] = gy
    xcur_ref[2:3, :] = gz
    outx_ref[0, 0, 0:1, :] = gx
    outx_ref[0, 0, 1:2, :] = gy
    outx_ref[0, 0, 2:3, :] = gz


def _sc_gather_call(table, flat_idx):
    total = flat_idx.shape[0]
    info = plsc.get_sparse_core_info()
    nw = info.num_cores * info.num_subcores
    kch = 16
    ch = kch * 128
    per_w = total // nw
    n_iter = per_w // ch
    assert per_w % ch == 0
    idx2d = flat_idx.reshape(total // 128, 128)

    mesh = plsc.VectorSubcoreMesh(core_axis_name="c", subcore_axis_name="s")

    @functools.partial(
        pl.kernel,
        mesh=mesh,
        out_type=jax.ShapeDtypeStruct((total, ROWW), jnp.float32),
        compiler_params=pltpu.CompilerParams(use_tc_tiling_on_sc=False),
        scratch_types=[
            pltpu.VMEM((kch, 128), jnp.int32),
            pltpu.VMEM((ch, ROWW), jnp.float32),
            pltpu.SemaphoreType.DMA,
        ],
    )
    def _k(table_hbm, idx_hbm, out_hbm, idx_v, rows_v, sem):
        wid = lax.axis_index("s") * info.num_cores + lax.axis_index("c")
        base = wid * per_w

        def body(i, carry):
            off = pl.multiple_of(base + i * ch, ch)
            pltpu.sync_copy(
                idx_hbm.at[pl.ds(pl.multiple_of(off // 128, kch), kch), :],
                idx_v)
            copies = []
            for j in range(kch):
                copies.append(pltpu.async_copy(
                    table_hbm.at[idx_v.at[j]],
                    rows_v.at[pl.ds(j * 128, 128), :],
                    sem,
                ))
            for cp in copies:
                cp.wait()
            pltpu.sync_copy(rows_v, out_hbm.at[pl.ds(off, ch), :])
            return carry

        lax.fori_loop(0, n_iter, body, 0)

    return _k(table, idx2d)


def kernel(point_seq):
    b, t, n, d = point_seq.shape
    assert t % TS == 0 and d == 3
    nseg = t // TS
    nq = n // 2
    nprob = 2 * b * nseg

    perm_key = jax.random.key(42)
    ridx = jnp.stack([
        jax.random.permutation(jax.random.fold_in(perm_key, i), n)[:nq]
        for i in range(nseg)
    ])

    b_arr = jnp.tile(jnp.repeat(jnp.arange(b), nseg), 2)
    seg_arr = jnp.tile(jnp.arange(nseg), 2 * b)
    dir_arr = jnp.repeat(jnp.arange(2), b * nseg)
    f0_arr = seg_arr * TS + jnp.where(dir_arr == 0, 0, TS - 1)

    q0 = point_seq[b_arr[:, None], f0_arr[:, None], ridx[seg_arr], :]
    q0 = jnp.transpose(q0, (0, 2, 1))

    bn = b * nseg

    def _key_map(p, s):
        dir_ = p // bn
        rem = p % bn
        b_ = rem // nseg
        seg = rem % nseg
        local = jnp.where(dir_ == 0, s, TS - 1 - s)
        return (b_, seg * TS + local, 0, 0)

    out_shapes = (
        jax.ShapeDtypeStruct((nprob, TS, K, nq), jnp.float32),
        jax.ShapeDtypeStruct((nprob, TS, K, nq), jnp.int32),
        jax.ShapeDtypeStruct((nprob, TS, 3, nq), jnp.float32),
    )

    grid = (nprob, TS)
    dist_all, idx_all, outx_all = pl.pallas_call(
        _cell,
        grid=grid,
        in_specs=[
            pl.BlockSpec((1, 3, nq), lambda p, s: (p, 0, 0)),
            pl.BlockSpec((1, 1, n, 3), _key_map),
        ],
        out_specs=[
            pl.BlockSpec((1, 1, K, nq), lambda p, s: (p, s, 0, 0)),
            pl.BlockSpec((1, 1, K, nq), lambda p, s: (p, s, 0, 0)),
            pl.BlockSpec((1, 1, 3, nq), lambda p, s: (p, s, 0, 0)),
        ],
        out_shape=out_shapes,
        scratch_shapes=[
            pltpu.VMEM((n, nq), jnp.float32),
            pltpu.VMEM((8, nq), jnp.float32),
        ],
        compiler_params=pltpu.CompilerParams(
            dimension_semantics=("parallel", "arbitrary"),
        ),
    )(q0, point_seq)

    def _split(x):
        x = x.reshape(2, b, nseg, *x.shape[1:])
        return x[0], x[1]

    def _asm_kq(x):
        f, bk = _split(x)
        bk = jnp.flip(bk, axis=2)
        y = jnp.concatenate([f, bk], axis=-1)
        return y.transpose(0, 1, 2, 4, 3).reshape(b, t, n, K)

    dist = _asm_kq(dist_all)
    idx = _asm_kq(idx_all)

    t_iota = jnp.arange(t, dtype=jnp.int32).reshape(1, t, 1, 1)
    b_iota = jnp.arange(b, dtype=jnp.int32).reshape(b, 1, 1, 1)
    n_iota = jnp.arange(n, dtype=jnp.int32).reshape(1, 1, n, 1)
    seg_base = (t_iota // TS) * TS
    featframe = jnp.where(
        n_iota < nq,
        jnp.maximum(t_iota - 1, seg_base),
        jnp.minimum(t_iota + 1, seg_base + TS - 1),
    )
    pts_rows = (b_iota * t + t_iota) * n + idx
    fe_rows = (b_iota * t + featframe) * n + idx
    flat_rows = jnp.concatenate(
        [pts_rows.reshape(-1), fe_rows.reshape(-1)])
    table = jnp.pad(point_seq.reshape(b * t * n, 3),
                    ((0, 0), (0, ROWW - 3)))
    gathered = _sc_gather_call(table, flat_rows)
    half = pts_rows.size
    pts = gathered[:half, 0:3].reshape(b, t, n, K, 3)
    pfe = gathered[half:, 0:3].reshape(b, t, n, K, 3)

    f, bk = _split(outx_all)
    outx = jnp.concatenate([f, bk], axis=-1)
    outx = outx.transpose(0, 1, 2, 4, 3).reshape(b, t, n, 3)

    return pts, pfe, dist, idx, idx, outx


# speedup vs baseline: 1.4941x; 1.1481x over previous
import functools

import jax
import jax.numpy as jnp
from jax import lax
from jax.experimental import pallas as pl
from jax.experimental.pallas import tpu as pltpu
from jax.experimental.pallas import tpu_sc as plsc

K = 16
TS = 8
ROWW = 16


def _cell(q0_ref, keys_ref,
          dist_ref, idx_ref, outx_ref,
          d2_ref, xcur_ref):
    s = pl.program_id(1)
    nk = keys_ref.shape[2]
    nq = q0_ref.shape[2]

    @pl.when(s == 0)
    def _init_chain():
        xcur_ref[0:3, :] = q0_ref[0]

    keys = keys_ref[0, 0]
    kx = keys[:, 0:1]
    ky = keys[:, 1:2]
    kz = keys[:, 2:3]
    dx = xcur_ref[0:1, :] - kx
    dy = xcur_ref[1:2, :] - ky
    dz = xcur_ref[2:3, :] - kz
    d2_ref[...] = dx * dx + dy * dy + dz * dz

    iota = jax.lax.broadcasted_iota(jnp.int32, (nk, nq), 0)

    for r in range(K):
        d2 = d2_ref[...]
        minv = jnp.min(d2, axis=0, keepdims=True)
        cand = jnp.where(d2 == minv, iota, jnp.int32(nk))
        mini = jnp.min(cand, axis=0, keepdims=True)
        if r < K - 1:
            d2_ref[...] = jnp.where(iota == mini, jnp.float32(jnp.inf), d2)
        dist_ref[0, 0, r, :] = minv[0]
        idx_ref[0, 0, r, :] = mini[0]
        if r == 0:
            onehot0 = iota == mini

    zero = jnp.float32(0.0)
    gx = jnp.sum(jnp.where(onehot0, kx, zero), axis=0, keepdims=True)
    gy = jnp.sum(jnp.where(onehot0, ky, zero), axis=0, keepdims=True)
    gz = jnp.sum(jnp.where(onehot0, kz, zero), axis=0, keepdims=True)
    xcur_ref[0:1, :] = gx
    xcur_ref[1:2, :] = gy
    xcur_ref[2:3, :] = gz
    outx_ref[0, 0, 0:1, :] = gx
    outx_ref[0, 0, 1:2, :] = gy
    outx_ref[0, 0, 2:3, :] = gz


def _sc_gather_call(table, flat_idx):
    total = flat_idx.shape[0]
    info = plsc.get_sparse_core_info()
    nw = info.num_cores * info.num_subcores
    kch = 16
    ch = kch * 128
    per_w = total // nw
    n_iter = per_w // ch
    assert per_w % ch == 0
    idx2d = flat_idx.reshape(total // 128, 128)

    mesh = plsc.VectorSubcoreMesh(core_axis_name="c", subcore_axis_name="s")

    @functools.partial(
        pl.kernel,
        mesh=mesh,
        out_type=jax.ShapeDtypeStruct((total, ROWW), jnp.float32),
        compiler_params=pltpu.CompilerParams(use_tc_tiling_on_sc=False),
        scratch_types=[
            pltpu.VMEM((kch, 128), jnp.int32),
            pltpu.VMEM((ch, ROWW), jnp.float32),
            pltpu.SemaphoreType.DMA,
        ],
    )
    def _k(table_hbm, idx_hbm, out_hbm, idx_v, rows_v, sem):
        wid = lax.axis_index("s") * info.num_cores + lax.axis_index("c")
        base = wid * per_w

        def body(i, carry):
            off = pl.multiple_of(base + i * ch, ch)
            pltpu.sync_copy(
                idx_hbm.at[pl.ds(pl.multiple_of(off // 128, kch), kch), :],
                idx_v)
            copies = []
            for j in range(kch):
                copies.append(pltpu.async_copy(
                    table_hbm.at[idx_v.at[j]],
                    rows_v.at[pl.ds(j * 128, 128), :],
                    sem,
                ))
            for cp in copies:
                cp.wait()
            pltpu.sync_copy(rows_v, out_hbm.at[pl.ds(off, ch), :])
            return carry

        lax.fori_loop(0, n_iter, body, 0)

    return _k(table, idx2d)


def kernel(point_seq):
    b, t, n, d = point_seq.shape
    assert t % TS == 0 and d == 3
    nseg = t // TS
    nq = n // 2
    nprob = 2 * b * nseg

    perm_key = jax.random.key(42)
    ridx = jnp.stack([
        jax.random.permutation(jax.random.fold_in(perm_key, i), n)[:nq]
        for i in range(nseg)
    ])

    b_arr = jnp.tile(jnp.repeat(jnp.arange(b), nseg), 2)
    seg_arr = jnp.tile(jnp.arange(nseg), 2 * b)
    dir_arr = jnp.repeat(jnp.arange(2), b * nseg)
    f0_arr = seg_arr * TS + jnp.where(dir_arr == 0, 0, TS - 1)

    q0 = point_seq[b_arr[:, None], f0_arr[:, None], ridx[seg_arr], :]
    q0 = jnp.transpose(q0, (0, 2, 1))

    bn = b * nseg

    def _key_map(p, s):
        dir_ = p // bn
        rem = p % bn
        b_ = rem // nseg
        seg = rem % nseg
        local = jnp.where(dir_ == 0, s, TS - 1 - s)
        return (b_, seg * TS + local, 0, 0)

    out_shapes = (
        jax.ShapeDtypeStruct((nprob, TS, K, nq), jnp.float32),
        jax.ShapeDtypeStruct((nprob, TS, K, nq), jnp.int32),
        jax.ShapeDtypeStruct((nprob, TS, 3, nq), jnp.float32),
    )

    grid = (nprob, TS)
    dist_all, idx_all, outx_all = pl.pallas_call(
        _cell,
        grid=grid,
        in_specs=[
            pl.BlockSpec((1, 3, nq), lambda p, s: (p, 0, 0)),
            pl.BlockSpec((1, 1, n, 3), _key_map),
        ],
        out_specs=[
            pl.BlockSpec((1, 1, K, nq), lambda p, s: (p, s, 0, 0)),
            pl.BlockSpec((1, 1, K, nq), lambda p, s: (p, s, 0, 0)),
            pl.BlockSpec((1, 1, 3, nq), lambda p, s: (p, s, 0, 0)),
        ],
        out_shape=out_shapes,
        scratch_shapes=[
            pltpu.VMEM((n, nq), jnp.float32),
            pltpu.VMEM((8, nq), jnp.float32),
        ],
        compiler_params=pltpu.CompilerParams(
            dimension_semantics=("parallel", "arbitrary"),
        ),
    )(q0, point_seq)

    def _split(x):
        x = x.reshape(2, b, nseg, *x.shape[1:])
        return x[0], x[1]

    def _asm_kq(x):
        f, bk = _split(x)
        bk = jnp.flip(bk, axis=2)
        y = jnp.concatenate([f, bk], axis=-1)
        return y.transpose(0, 1, 2, 4, 3).reshape(b, t, n, K)

    dist = _asm_kq(dist_all)
    idx = _asm_kq(idx_all)

    t_ar = jnp.arange(t, dtype=jnp.int32)
    seg_base = (t_ar // TS) * TS
    ff_fwd = jnp.maximum(t_ar - 1, seg_base)
    ff_bwd = jnp.minimum(t_ar + 1, seg_base + TS - 1)
    pcoord = point_seq.reshape(b * t * n, 3)
    fe_fwd = point_seq[:, ff_fwd].reshape(b * t * n, 3)
    fe_bwd = point_seq[:, ff_bwd].reshape(b * t * n, 3)
    zpad = jnp.zeros((b * t * n, ROWW - 6), jnp.float32)
    table = jnp.concatenate([
        jnp.concatenate([pcoord, fe_fwd, zpad], axis=1),
        jnp.concatenate([pcoord, fe_bwd, zpad], axis=1),
    ], axis=0)

    t_iota = t_ar.reshape(1, t, 1, 1)
    b_iota = jnp.arange(b, dtype=jnp.int32).reshape(b, 1, 1, 1)
    n_iota = jnp.arange(n, dtype=jnp.int32).reshape(1, 1, n, 1)
    v_iota = (n_iota >= nq).astype(jnp.int32)
    rows = ((v_iota * b + b_iota) * t + t_iota) * n + idx
    gathered = _sc_gather_call(table, rows.reshape(-1))
    pts = gathered[:, 0:3].reshape(b, t, n, K, 3)
    pfe = gathered[:, 3:6].reshape(b, t, n, K, 3)

    f, bk = _split(outx_all)
    outx = jnp.concatenate([f, bk], axis=-1)
    outx = outx.transpose(0, 1, 2, 4, 3).reshape(b, t, n, 3)

    return pts, pfe, dist, idx, idx, outx
